# baseline (speedup 1.0000x reference)
import functools

import jax
import jax.numpy as jnp
from jax import lax
from jax.experimental import pallas as pl
from jax.experimental.pallas import tpu as pltpu
from jax.experimental.pallas import tpu_sc as plsc

B, F, V, D = 4096, 26, 100000, 32
H1, H2 = 512, 256
FD = F * D
BF = B * F
EPS = 1e-5

_L = 16
_NC, _NS = 2, 16
_NW = _NC * _NS
_NPW = BF // _NW
_G = 128
_NG = _NPW // _G

_HIGH = lax.Precision.HIGHEST


def _sc_gather_body(w2_hbm, w1_hbm, xi_hbm, e2_hbm, e1_hbm,
                    idx_v, rows_v, e1_v, sem2, sem1):
    wid = lax.axis_index("s") * _NC + lax.axis_index("c")
    base = wid * _NPW
    pltpu.sync_copy(xi_hbm.at[wid], idx_v)

    def _flatten(g, carry):
        for c in range(_G // _L):
            q = lax.iota(jnp.int32, _L) + (g * _G + c * _L)
            f = lax.rem(q, F)
            sl = pl.ds(c * _L, _L)
            idx_v[g, sl] = idx_v[g, sl] + f * V
        return carry
    lax.fori_loop(0, _NG, _flatten, 0)

    copies = []
    for g in range(_NG):
        copies.append(pltpu.async_copy(
            w2_hbm.at[idx_v.at[g]], rows_v.at[pl.ds(g * _G, _G)], sem2))
        copies.append(pltpu.async_copy(
            w1_hbm.at[idx_v.at[g]], e1_v.at[g], sem1))
    for cp in copies:
        cp.wait()

    pltpu.sync_copy(rows_v, e2_hbm.at[pl.ds(base, _NPW)])
    pltpu.sync_copy(e1_v, e1_hbm.at[wid])


@functools.cache
def _sc_gather():
    return pl.kernel(
        _sc_gather_body,
        out_type=(jax.ShapeDtypeStruct((BF, D), jnp.float32),
                  jax.ShapeDtypeStruct((_NW, _NG, _G), jnp.float32)),
        mesh=plsc.VectorSubcoreMesh(core_axis_name="c", subcore_axis_name="s"),
        compiler_params=pltpu.CompilerParams(use_tc_tiling_on_sc=False),
        scratch_types=[
            pltpu.VMEM((_NG, _G), jnp.int32),
            pltpu.VMEM((_NPW, D), jnp.float32),
            pltpu.VMEM((_NG, _G), jnp.float32),
            pltpu.SemaphoreType.DMA,
            pltpu.SemaphoreType.DMA,
        ],
    )


_C = 512
_NCH = B // _C


def _tc1_body(e2_ref, e1_ref, xv_ref, w1_ref, b1_ref,
              h1_ref, fmv_ref, s1_ref, ss1_ref):
    i = pl.program_id(0)
    xv = xv_ref[...]
    jj = lax.broadcasted_iota(jnp.int32, (F, FD), 1)
    ff = lax.broadcasted_iota(jnp.int32, (F, FD), 0)
    Rm = (jj // D == ff).astype(jnp.float32)
    xvr = lax.dot_general(xv, Rm, (((1,), (0,)), ((), ())), precision=_HIGH)
    x = e2_ref[...] * xvr
    rr = lax.broadcasted_iota(jnp.int32, (FD, D), 0)
    cc = lax.broadcasted_iota(jnp.int32, (FD, D), 1)
    Sm = (rr % D == cc).astype(jnp.float32)
    xs = lax.dot_general(x, Sm, (((1,), (0,)), ((), ())), precision=_HIGH)
    x2s = lax.dot_general(x * x, Sm, (((1,), (0,)), ((), ())),
                          precision=_HIGH)
    fm2 = 0.5 * (xs * xs - x2s)
    fmv_ref[...] = (jnp.sum(e1_ref[...] * xv, axis=1, keepdims=True)
                    + jnp.sum(fm2, axis=1, keepdims=True))
    h1 = lax.dot_general(x, w1_ref[...], (((1,), (1,)), ((), ())),
                         precision=_HIGH,
                         preferred_element_type=jnp.float32) + b1_ref[...]
    h1_ref[...] = h1

    @pl.when(i == 0)
    def _():
        s1_ref[...] = jnp.zeros_like(s1_ref)
        ss1_ref[...] = jnp.zeros_like(ss1_ref)
    s1_ref[...] += jnp.sum(h1, axis=0, keepdims=True)
    ss1_ref[...] += jnp.sum(h1 * h1, axis=0, keepdims=True)


def _tc2_body(h1_ref, s1_ref, ss1_ref, g1_ref, bb1_ref, w2_ref, b2_ref,
              h2_ref, s2_ref, ss2_ref):
    i = pl.program_id(0)
    mu1 = s1_ref[...] * (1.0 / B)
    var1 = ss1_ref[...] * (1.0 / B) - mu1 * mu1
    a1 = g1_ref[...] * lax.rsqrt(var1 + EPS)
    c1 = bb1_ref[...] - mu1 * a1
    bn1 = h1_ref[...] * a1 + c1
    h2 = lax.dot_general(bn1, w2_ref[...], (((1,), (1,)), ((), ())),
                         precision=_HIGH,
                         preferred_element_type=jnp.float32) + b2_ref[...]
    h2_ref[...] = h2

    @pl.when(i == 0)
    def _():
        s2_ref[...] = jnp.zeros_like(s2_ref)
        ss2_ref[...] = jnp.zeros_like(ss2_ref)
    s2_ref[...] += jnp.sum(h2, axis=0, keepdims=True)
    ss2_ref[...] += jnp.sum(h2 * h2, axis=0, keepdims=True)


def _tc3_body(h2_ref, s2_ref, ss2_ref, g2_ref, bb2_ref, fmv_ref, bias_ref,
              out_ref):
    mu2 = s2_ref[...] * (1.0 / B)
    var2 = ss2_ref[...] * (1.0 / B) - mu2 * mu2
    a2 = g2_ref[...] * lax.rsqrt(var2 + EPS)
    c2 = bb2_ref[...] - mu2 * a2
    deep = jnp.sum(h2_ref[...] * a2, axis=1, keepdims=True) + jnp.sum(c2)
    out_ref[...] = fmv_ref[...] + deep + bias_ref[0, 0]


def _dense(e2, e1, xv, lin1_W, lin1_b, bn1_g, bn1_b,
           lin2_W, lin2_b, bn2_g, bn2_b, bias):
    row = lambda v: v.reshape(1, -1)
    h1, fmv, s1, ss1 = pl.pallas_call(
        _tc1_body,
        grid=(_NCH,),
        in_specs=[
            pl.BlockSpec((_C, FD), lambda i: (i, 0)),
            pl.BlockSpec((_C, F), lambda i: (i, 0)),
            pl.BlockSpec((_C, F), lambda i: (i, 0)),
            pl.BlockSpec((H1, FD), lambda i: (0, 0)),
            pl.BlockSpec((1, H1), lambda i: (0, 0)),
        ],
        out_specs=[
            pl.BlockSpec((_C, H1), lambda i: (i, 0)),
            pl.BlockSpec((_C, 1), lambda i: (i, 0)),
            pl.BlockSpec((1, H1), lambda i: (0, 0)),
            pl.BlockSpec((1, H1), lambda i: (0, 0)),
        ],
        out_shape=[
            jax.ShapeDtypeStruct((B, H1), jnp.float32),
            jax.ShapeDtypeStruct((B, 1), jnp.float32),
            jax.ShapeDtypeStruct((1, H1), jnp.float32),
            jax.ShapeDtypeStruct((1, H1), jnp.float32),
        ],
    )(e2, e1, xv, lin1_W, row(lin1_b))

    h2, s2, ss2 = pl.pallas_call(
        _tc2_body,
        grid=(_NCH,),
        in_specs=[
            pl.BlockSpec((_C, H1), lambda i: (i, 0)),
            pl.BlockSpec((1, H1), lambda i: (0, 0)),
            pl.BlockSpec((1, H1), lambda i: (0, 0)),
            pl.BlockSpec((1, H1), lambda i: (0, 0)),
            pl.BlockSpec((1, H1), lambda i: (0, 0)),
            pl.BlockSpec((H2, H1), lambda i: (0, 0)),
            pl.BlockSpec((1, H2), lambda i: (0, 0)),
        ],
        out_specs=[
            pl.BlockSpec((_C, H2), lambda i: (i, 0)),
            pl.BlockSpec((1, H2), lambda i: (0, 0)),
            pl.BlockSpec((1, H2), lambda i: (0, 0)),
        ],
        out_shape=[
            jax.ShapeDtypeStruct((B, H2), jnp.float32),
            jax.ShapeDtypeStruct((1, H2), jnp.float32),
            jax.ShapeDtypeStruct((1, H2), jnp.float32),
        ],
    )(h1, s1, ss1, row(bn1_g), row(bn1_b), lin2_W, row(lin2_b))

    total = pl.pallas_call(
        _tc3_body,
        in_specs=[
            pl.BlockSpec((B, H2), lambda: (0, 0)),
            pl.BlockSpec((1, H2), lambda: (0, 0)),
            pl.BlockSpec((1, H2), lambda: (0, 0)),
            pl.BlockSpec((1, H2), lambda: (0, 0)),
            pl.BlockSpec((1, H2), lambda: (0, 0)),
            pl.BlockSpec((B, 1), lambda: (0, 0)),
            pl.BlockSpec((1, 1), lambda: (0, 0)),
        ],
        out_specs=pl.BlockSpec((B, 1), lambda: (0, 0)),
        out_shape=jax.ShapeDtypeStruct((B, 1), jnp.float32),
    )(h2, s2, ss2, row(bn2_g), row(bn2_b), fmv, bias.reshape(1, 1))
    return total.reshape(B)


def kernel(Xi, Xv, W1, W2, lin1_W, lin1_b, bn1_g, bn1_b,
           lin2_W, lin2_b, bn2_g, bn2_b, bias):
    xi = Xi.reshape(_NW, _NG, _G).astype(jnp.int32)
    e2_flat, e1_flat = _sc_gather()(
        W2.reshape(F * V, D), W1.reshape(F * V), xi)
    e2 = e2_flat.reshape(B, FD)
    e1 = e1_flat.reshape(B, F)
    return _dense(e2, e1, Xv, lin1_W, lin1_b, bn1_g, bn1_b,
                  lin2_W, lin2_b, bn2_g, bn2_b, bias)


# speedup vs baseline: 1.4245x; 1.4245x over previous
import functools

import jax
import jax.numpy as jnp
from jax import lax
from jax.experimental import pallas as pl
from jax.experimental.pallas import tpu as pltpu
from jax.experimental.pallas import tpu_sc as plsc

B, F, V, D = 4096, 26, 100000, 32
H1, H2 = 512, 256
FD = F * D
EPS = 1e-5

_L = 16
_NC, _NS = 2, 16
_NW = _NC * _NS
_K2 = FD // _NW

_HIGH = lax.Precision.HIGHEST


def _sc_gather_body(w2_hbm, w1_hbm, xi_hbm, x_hbm, e1_hbm,
                    idx_v, out_v, sem):
    wid = lax.axis_index("s") * _NC + lax.axis_index("c")
    _G = 128
    _NGC = B // _G

    def gather_row(table_flat, base):
        def flat(i, carry):
            off = pl.multiple_of(i * _L, _L)
            idx_v[pl.ds(off, _L)] = idx_v[pl.ds(off, _L)] + base
            return carry
        lax.fori_loop(0, B // _L, flat, 0)
        cps = [pltpu.async_copy(table_flat.at[idx_v.at[pl.ds(j * _G, _G)]],
                                out_v.at[pl.ds(j * _G, _G)], sem)
               for j in range(_NGC)]
        for cp in cps:
            cp.wait()

    @pl.when(wid < F)
    def _():
        pltpu.sync_copy(xi_hbm.at[wid], idx_v)
        gather_row(w1_hbm, wid * V)
        pltpu.sync_copy(out_v, e1_hbm.at[wid])

    def unit(k, carry):
        u = wid + k * _NW
        f = u // D
        pltpu.sync_copy(xi_hbm.at[f], idx_v)
        gather_row(w2_hbm, u * V)
        pltpu.sync_copy(out_v, x_hbm.at[u])
        return carry
    lax.fori_loop(0, _K2, unit, 0)


@functools.cache
def _sc_gather():
    return pl.kernel(
        _sc_gather_body,
        out_type=(jax.ShapeDtypeStruct((FD, B), jnp.float32),
                  jax.ShapeDtypeStruct((F, B), jnp.float32)),
        mesh=plsc.VectorSubcoreMesh(core_axis_name="c", subcore_axis_name="s"),
        compiler_params=pltpu.CompilerParams(use_tc_tiling_on_sc=False),
        scratch_types=[
            pltpu.VMEM((B,), jnp.int32),
            pltpu.VMEM((B,), jnp.float32),
            pltpu.SemaphoreType.DMA,
        ],
    )


_C = 512
_NCH = B // _C


def _tc1_body(x_ref, e1_ref, xv_ref, w1_ref, b1_ref,
              h1_ref, fmv_ref, s1_ref, ss1_ref):
    i = pl.program_id(0)
    xv = xv_ref[...]
    jj = lax.broadcasted_iota(jnp.int32, (FD, F), 0)
    ff = lax.broadcasted_iota(jnp.int32, (FD, F), 1)
    R2 = (jj // D == ff).astype(jnp.float32)
    xvr = lax.dot_general(R2, xv, (((1,), (0,)), ((), ())), precision=_HIGH)
    x = x_ref[...] * xvr
    cc = lax.broadcasted_iota(jnp.int32, (D, FD), 0)
    rr = lax.broadcasted_iota(jnp.int32, (D, FD), 1)
    S2 = (rr % D == cc).astype(jnp.float32)
    xs = lax.dot_general(S2, x, (((1,), (0,)), ((), ())), precision=_HIGH)
    x2s = lax.dot_general(S2, x * x, (((1,), (0,)), ((), ())),
                          precision=_HIGH)
    fm2 = 0.5 * (xs * xs - x2s)
    fmv_ref[...] = (jnp.sum(e1_ref[...] * xv, axis=0, keepdims=True)
                    + jnp.sum(fm2, axis=0, keepdims=True))
    h1 = lax.dot_general(w1_ref[...], x, (((1,), (0,)), ((), ())),
                         precision=_HIGH,
                         preferred_element_type=jnp.float32) + b1_ref[...]
    h1_ref[...] = h1

    @pl.when(i == 0)
    def _():
        s1_ref[...] = jnp.zeros_like(s1_ref)
        ss1_ref[...] = jnp.zeros_like(ss1_ref)
    s1_ref[...] += jnp.sum(h1, axis=1, keepdims=True)
    ss1_ref[...] += jnp.sum(h1 * h1, axis=1, keepdims=True)


def _tc2_body(h1_ref, s1_ref, ss1_ref, g1_ref, bb1_ref, w2_ref, b2_ref,
              h2_ref, s2_ref, ss2_ref):
    i = pl.program_id(0)
    mu1 = s1_ref[...] * (1.0 / B)
    var1 = ss1_ref[...] * (1.0 / B) - mu1 * mu1
    a1 = g1_ref[...] * lax.rsqrt(var1 + EPS)
    c1 = bb1_ref[...] - mu1 * a1
    bn1 = h1_ref[...] * a1 + c1
    h2 = lax.dot_general(w2_ref[...], bn1, (((1,), (0,)), ((), ())),
                         precision=_HIGH,
                         preferred_element_type=jnp.float32) + b2_ref[...]
    h2_ref[...] = h2

    @pl.when(i == 0)
    def _():
        s2_ref[...] = jnp.zeros_like(s2_ref)
        ss2_ref[...] = jnp.zeros_like(ss2_ref)
    s2_ref[...] += jnp.sum(h2, axis=1, keepdims=True)
    ss2_ref[...] += jnp.sum(h2 * h2, axis=1, keepdims=True)


def _tc3_body(h2_ref, s2_ref, ss2_ref, g2_ref, bb2_ref, fmv_ref, bias_ref,
              out_ref):
    mu2 = s2_ref[...] * (1.0 / B)
    var2 = ss2_ref[...] * (1.0 / B) - mu2 * mu2
    a2 = g2_ref[...] * lax.rsqrt(var2 + EPS)
    c2 = bb2_ref[...] - mu2 * a2
    deep = jnp.sum(h2_ref[...] * a2, axis=0, keepdims=True) + jnp.sum(c2)
    out_ref[...] = fmv_ref[...] + deep + bias_ref[0, 0]


def _dense(xT, e1T, xvT, lin1_W, lin1_b, bn1_g, bn1_b,
           lin2_W, lin2_b, bn2_g, bn2_b, bias):
    col = lambda v: v.reshape(-1, 1)
    h1, fmv, s1, ss1 = pl.pallas_call(
        _tc1_body,
        grid=(_NCH,),
        in_specs=[
            pl.BlockSpec((FD, _C), lambda i: (0, i)),
            pl.BlockSpec((F, _C), lambda i: (0, i)),
            pl.BlockSpec((F, _C), lambda i: (0, i)),
            pl.BlockSpec((H1, FD), lambda i: (0, 0)),
            pl.BlockSpec((H1, 1), lambda i: (0, 0)),
        ],
        out_specs=[
            pl.BlockSpec((H1, _C), lambda i: (0, i)),
            pl.BlockSpec((1, _C), lambda i: (0, i)),
            pl.BlockSpec((H1, 1), lambda i: (0, 0)),
            pl.BlockSpec((H1, 1), lambda i: (0, 0)),
        ],
        out_shape=[
            jax.ShapeDtypeStruct((H1, B), jnp.float32),
            jax.ShapeDtypeStruct((1, B), jnp.float32),
            jax.ShapeDtypeStruct((H1, 1), jnp.float32),
            jax.ShapeDtypeStruct((H1, 1), jnp.float32),
        ],
    )(xT, e1T, xvT, lin1_W, col(lin1_b))

    h2, s2, ss2 = pl.pallas_call(
        _tc2_body,
        grid=(_NCH,),
        in_specs=[
            pl.BlockSpec((H1, _C), lambda i: (0, i)),
            pl.BlockSpec((H1, 1), lambda i: (0, 0)),
            pl.BlockSpec((H1, 1), lambda i: (0, 0)),
            pl.BlockSpec((H1, 1), lambda i: (0, 0)),
            pl.BlockSpec((H1, 1), lambda i: (0, 0)),
            pl.BlockSpec((H2, H1), lambda i: (0, 0)),
            pl.BlockSpec((H2, 1), lambda i: (0, 0)),
        ],
        out_specs=[
            pl.BlockSpec((H2, _C), lambda i: (0, i)),
            pl.BlockSpec((H2, 1), lambda i: (0, 0)),
            pl.BlockSpec((H2, 1), lambda i: (0, 0)),
        ],
        out_shape=[
            jax.ShapeDtypeStruct((H2, B), jnp.float32),
            jax.ShapeDtypeStruct((H2, 1), jnp.float32),
            jax.ShapeDtypeStruct((H2, 1), jnp.float32),
        ],
    )(h1, s1, ss1, col(bn1_g), col(bn1_b), lin2_W, col(lin2_b))

    total = pl.pallas_call(
        _tc3_body,
        in_specs=[
            pl.BlockSpec((H2, B), lambda: (0, 0)),
            pl.BlockSpec((H2, 1), lambda: (0, 0)),
            pl.BlockSpec((H2, 1), lambda: (0, 0)),
            pl.BlockSpec((H2, 1), lambda: (0, 0)),
            pl.BlockSpec((H2, 1), lambda: (0, 0)),
            pl.BlockSpec((1, B), lambda: (0, 0)),
            pl.BlockSpec((1, 1), lambda: (0, 0)),
        ],
        out_specs=pl.BlockSpec((1, B), lambda: (0, 0)),
        out_shape=jax.ShapeDtypeStruct((1, B), jnp.float32),
    )(h2, s2, ss2, col(bn2_g), col(bn2_b), fmv, bias.reshape(1, 1))
    return total.reshape(B)


def kernel(Xi, Xv, W1, W2, lin1_W, lin1_b, bn1_g, bn1_b,
           lin2_W, lin2_b, bn2_g, bn2_b, bias):
    w2v = jnp.transpose(W2, (0, 2, 1)).reshape(FD * V)
    w1v = jnp.transpose(W1, (2, 0, 1)).reshape(F * V)
    xiT = jnp.transpose(Xi.reshape(B, F)).astype(jnp.int32)
    xT, e1T = _sc_gather()(w2v, w1v, xiT)
    xvT = jnp.transpose(Xv)
    return _dense(xT, e1T, xvT, lin1_W, lin1_b, bn1_g, bn1_b,
                  lin2_W, lin2_b, bn2_g, bn2_b, bias)
